# baseline (speedup 1.0000x reference)
import functools

import numpy as np
import jax
import jax.numpy as jnp
from jax import lax
from jax.experimental import pallas as pl
from jax.experimental.pallas import tpu as pltpu
from jax.experimental.pallas import tpu_sc as plsc

_VOCAB = 1000000
_EMBED = 64
_SEQ = 200
_BATCH = 4096
_N = _BATCH * _SEQ

_NW = 32
_JB = _BATCH // 128
_UNITS = _SEQ * _JB
_PW = _UNITS // _NW
_GROUPS = _PW // 8
_L = 16
_DT = _EMBED // 8


def _make_pe():
    pos = np.arange(_SEQ, dtype=np.float32)[:, None]
    div = np.exp(np.arange(0, _EMBED, 2, dtype=np.float32)
                 * -(np.log(10000.0) / _EMBED))
    pe = np.zeros((_SEQ, _EMBED), np.float32)
    pe[:, 0::2] = np.sin(pos * div)
    pe[:, 1::2] = np.cos(pos * div)
    return pe


_PE = _make_pe()


def _sc_embed(seq_lin, table, pe):
    mesh = plsc.VectorSubcoreMesh(core_axis_name="c", subcore_axis_name="s")

    @functools.partial(
        pl.kernel,
        mesh=mesh,
        out_type=jax.ShapeDtypeStruct((_SEQ, _DT, _JB, 8, 128), jnp.float32),
        scratch_types=[
            pltpu.VMEM((8, 128), jnp.int32),
            pltpu.VMEM((8, 128, _EMBED), jnp.float32),
            pltpu.VMEM((2, _DT, 8, 128), jnp.float32),
            pltpu.VMEM((_SEQ, _EMBED), jnp.float32),
            pltpu.SemaphoreType.DMA,
            pltpu.SemaphoreType.DMA,
        ],
        compiler_params=pltpu.CompilerParams(use_tc_tiling_on_sc=False,
                                             needs_layout_passes=False),
    )
    def k(seq_hbm, tab_hbm, pe_hbm, out_hbm, idx_v, g, st, pe_v, gsem, wsem):
        wid = lax.axis_index("s") * 2 + lax.axis_index("c")
        base_r = wid * _PW
        pltpu.sync_copy(pe_hbm, pe_v)
        tok_vecs = [lax.iota(jnp.int32, _L) + (h * _L) for h in range(8)]

        def unit_compute(u, r):
            s = r // _JB
            j = lax.rem(r, _JB)
            s_splat = jnp.full((_L,), s, jnp.int32)

            def d_body(d, carry2):
                d_splat = jnp.full((_L,), d, jnp.int32)
                pe_val = plsc.load_gather(pe_v, [s_splat, d_splat])
                vs = [plsc.load_gather(g.at[u], [tok_vecs[h], d_splat])
                      for h in range(8)]
                vs = [v + pe_val for v in vs]
                for h in range(8):
                    st[u % 2, d // 8, lax.rem(d, 8), pl.ds(h * _L, _L)] = vs[h]
                return carry2

            lax.fori_loop(0, _EMBED, d_body, 0, unroll=2)
            return s, j

        def write(u, s, j):
            return [pltpu.async_copy(st.at[u % 2], out_hbm.at[s, :, j], wsem)]

        def group_body(gi, carry):
            gr = base_r + gi * 8
            gr8 = pl.multiple_of(gr, 8)
            pltpu.sync_copy(seq_hbm.at[pl.ds(gr8, 8)], idx_v)
            cps = [pltpu.async_copy(tab_hbm.at[idx_v.at[u]], g.at[u], gsem)
                   for u in range(8)]
            for cp in cps:
                cp.wait()
            wcs = {}
            for u in range(8):
                if u - 2 in wcs:
                    for wc in wcs.pop(u - 2):
                        wc.wait()
                s, j = unit_compute(u, gr + u)
                wcs[u] = write(u, s, j)
            for ws in wcs.values():
                for wc in ws:
                    wc.wait()
            return carry

        lax.fori_loop(0, _GROUPS, group_body, 0)

    return k(seq_lin, table, pe)


@jax.jit
def kernel(sequence, table):
    seq_lin = jnp.transpose(sequence).reshape(_UNITS, 128).astype(jnp.int32)
    pe = jnp.asarray(_PE)
    lin5 = _sc_embed(seq_lin, table, pe)
    return lin5.transpose(2, 4, 0, 1, 3).reshape(_BATCH, _SEQ, _EMBED)


# speedup vs baseline: 1.0001x; 1.0001x over previous
import functools

import numpy as np
import jax
import jax.numpy as jnp
from jax import lax
from jax.experimental import pallas as pl
from jax.experimental.pallas import tpu as pltpu
from jax.experimental.pallas import tpu_sc as plsc

_VOCAB = 1000000
_EMBED = 64
_SEQ = 200
_BATCH = 4096
_N = _BATCH * _SEQ

_NW = 32
_JB = _BATCH // 128
_UNITS = _SEQ * _JB
_PW = _UNITS // _NW
_GROUPS = _PW // 8
_L = 16
_DT = _EMBED // 8


def _make_pe():
    pos = np.arange(_SEQ, dtype=np.float32)[:, None]
    div = np.exp(np.arange(0, _EMBED, 2, dtype=np.float32)
                 * -(np.log(10000.0) / _EMBED))
    pe = np.zeros((_SEQ, _EMBED), np.float32)
    pe[:, 0::2] = np.sin(pos * div)
    pe[:, 1::2] = np.cos(pos * div)
    return pe


_PE = _make_pe()


def _sc_embed(seq_lin, table, pe):
    mesh = plsc.VectorSubcoreMesh(core_axis_name="c", subcore_axis_name="s")

    @functools.partial(
        pl.kernel,
        mesh=mesh,
        out_type=jax.ShapeDtypeStruct((_SEQ, _DT, _JB, 8, 128), jnp.float32),
        scratch_types=[
            pltpu.VMEM((8, 128), jnp.int32),
            pltpu.VMEM((1024, _EMBED), jnp.float32),
            pltpu.VMEM((2, _DT, 8, 128), jnp.float32),
            pltpu.VMEM((_SEQ, _EMBED), jnp.float32),
            pltpu.SemaphoreType.DMA,
            pltpu.SemaphoreType.DMA,
        ],
        compiler_params=pltpu.CompilerParams(use_tc_tiling_on_sc=False,
                                             needs_layout_passes=False),
    )
    def k(seq_hbm, tab_hbm, pe_hbm, out_hbm, idx_v, g, st, pe_v, gsem, wsem):
        wid = lax.axis_index("s") * 2 + lax.axis_index("c")
        base_r = wid * _PW
        pltpu.sync_copy(pe_hbm, pe_v)
        tok_vecs = [lax.iota(jnp.int32, _L) + (h * _L) for h in range(8)]

        def unit_compute(u, r):
            s = r // _JB
            j = lax.rem(r, _JB)
            s_splat = jnp.full((_L,), s, jnp.int32)
            tok_vecs_u = [tv + (u * 128) for tv in tok_vecs]

            def d_body(d, carry2):
                d_splat = jnp.full((_L,), d, jnp.int32)
                pe_val = plsc.load_gather(pe_v, [s_splat, d_splat])
                vs = [plsc.load_gather(g, [tok_vecs_u[h], d_splat])
                      for h in range(8)]
                vs = [v + pe_val for v in vs]
                for h in range(8):
                    st[u % 2, d // 8, lax.rem(d, 8), pl.ds(h * _L, _L)] = vs[h]
                return carry2

            lax.fori_loop(0, _EMBED, d_body, 0, unroll=2)
            return s, j

        def write(u, s, j):
            return [pltpu.async_copy(st.at[u % 2], out_hbm.at[s, :, j], wsem)]

        def group_body(gi, carry):
            gr = base_r + gi * 8
            gr8 = pl.multiple_of(gr, 8)
            pltpu.sync_copy(seq_hbm.at[pl.ds(gr8, 8)], idx_v)
            cps = [pltpu.async_copy(tab_hbm.at[idx_v.at[u]],
                                    g.at[pl.ds(u * 128, 128)], gsem)
                   for u in range(8)]
            for cp in cps:
                cp.wait()
            wcs = {}
            for u in range(8):
                if u - 2 in wcs:
                    for wc in wcs.pop(u - 2):
                        wc.wait()
                s, j = unit_compute(u, gr + u)
                wcs[u] = write(u, s, j)
            for ws in wcs.values():
                for wc in ws:
                    wc.wait()
            return carry

        lax.fori_loop(0, _GROUPS, group_body, 0)

    return k(seq_lin, table, pe)


@jax.jit
def kernel(sequence, table):
    seq_lin = jnp.transpose(sequence).reshape(_UNITS, 128).astype(jnp.int32)
    pe = jnp.asarray(_PE)
    lin5 = _sc_embed(seq_lin, table, pe)
    return lin5.transpose(2, 4, 0, 1, 3).reshape(_BATCH, _SEQ, _EMBED)


# speedup vs baseline: 1.2996x; 1.2994x over previous
import functools

import numpy as np
import jax
import jax.numpy as jnp
from jax import lax
from jax.experimental import pallas as pl
from jax.experimental.pallas import tpu as pltpu
from jax.experimental.pallas import tpu_sc as plsc

_VOCAB = 1000000
_EMBED = 64
_SEQ = 200
_BATCH = 4096

_NW = 32
_JB = _BATCH // 128
_UNITS = _SEQ * _JB
_PW = _UNITS // _NW
_GROUPS = _PW // 8
_L = 16
_DT = _EMBED // 8
_GS = 65


def _make_pe():
    pos = np.arange(_SEQ, dtype=np.float32)[:, None]
    div = np.exp(np.arange(0, _EMBED, 2, dtype=np.float32)
                 * -(np.log(10000.0) / _EMBED))
    pe = np.zeros((_SEQ, _EMBED), np.float32)
    pe[:, 0::2] = np.sin(pos * div)
    pe[:, 1::2] = np.cos(pos * div)
    return pe


_PES = np.repeat(_make_pe()[:, :, None], _L, axis=2)


def _sc_embed(seq_lin, table, pes):
    mesh = plsc.VectorSubcoreMesh(core_axis_name="c", subcore_axis_name="s")

    @functools.partial(
        pl.kernel,
        mesh=mesh,
        out_type=jax.ShapeDtypeStruct((_SEQ, _DT, _JB, 8, 128), jnp.float32),
        scratch_types=[
            pltpu.VMEM((8, 128), jnp.int32),
            pltpu.VMEM((1024, _EMBED), jnp.float32),
            pltpu.VMEM((128, _GS), jnp.float32),
            pltpu.VMEM((2, _DT, 8, 128), jnp.float32),
            pltpu.VMEM((_EMBED, _L), jnp.float32),
            pltpu.SemaphoreType.DMA,
            pltpu.SemaphoreType.DMA,
        ],
        compiler_params=pltpu.CompilerParams(use_tc_tiling_on_sc=False,
                                             needs_layout_passes=False),
    )
    def k(seq_hbm, tab_hbm, pes_hbm, out_hbm, idx_v, g, g65, st, pes_v, gsem,
          wsem):
        wid = lax.axis_index("s") * 2 + lax.axis_index("c")
        base_r = wid * _PW
        tok_vecs = [lax.iota(jnp.int32, _L) + (h * _L) for h in range(8)]

        def unit_compute(u, j):
            def stage_body(r, carry2):
                for kk in range(_EMBED // _L):
                    sl = pl.ds(kk * _L, _L)
                    g65[r, sl] = g[u * 128 + r, sl]
                return carry2

            lax.fori_loop(0, 128, stage_body, 0, unroll=8)

            def d_body(d, carry2):
                pe_val = pes_v[d]
                d_splat = jnp.full((_L,), d, jnp.int32)
                vs = [plsc.load_gather(g65, [tok_vecs[h], d_splat])
                      for h in range(8)]
                vs = [v + pe_val for v in vs]
                for h in range(8):
                    st[u % 2, d // 8, lax.rem(d, 8), pl.ds(h * _L, _L)] = vs[h]
                return carry2

            lax.fori_loop(0, _EMBED, d_body, 0, unroll=2)

        def group_body(gi, carry):
            gr = base_r + gi * 8
            gr8 = pl.multiple_of(gr, 8)
            s = gr // _JB
            j0 = lax.rem(gr, _JB)
            pltpu.sync_copy(seq_hbm.at[pl.ds(gr8, 8)], idx_v)
            cps = [pltpu.async_copy(tab_hbm.at[idx_v.at[u]],
                                    g.at[pl.ds(u * 128, 128)], gsem)
                   for u in range(8)]
            pltpu.sync_copy(pes_hbm.at[s], pes_v)
            wcs = {}
            for u in range(8):
                cps[u].wait()
                if u - 2 in wcs:
                    wcs.pop(u - 2).wait()
                unit_compute(u, j0 + u)
                wcs[u] = pltpu.async_copy(st.at[u % 2],
                                          out_hbm.at[s, :, j0 + u], wsem)
            for wc in wcs.values():
                wc.wait()
            return carry

        lax.fori_loop(0, _GROUPS, group_body, 0)

    return k(seq_lin, table, pes)


@jax.jit
def kernel(sequence, table):
    seq_lin = jnp.transpose(sequence).reshape(_UNITS, 128).astype(jnp.int32)
    pes = jnp.asarray(_PES)
    lin5 = _sc_embed(seq_lin, table, pes)
    return lin5.transpose(2, 4, 0, 1, 3).reshape(_BATCH, _SEQ, _EMBED)
